# baseline (speedup 1.0000x reference)
import functools

import jax
import jax.numpy as jnp
from jax import lax
from jax.experimental import pallas as pl
from jax.experimental.pallas import tpu as pltpu
from jax.experimental.pallas import tpu_sc as plsc

B = 8192
D = 1024
DB = 128
NC = 2
NS = 16
L = 16
NW = NC * NS
ROWS_PER_W = B // NW

MARK_AX, OP_AND, OP_OR, OP_XOR = 0, 1, 2, 3
ALU_LO, ALU_HI, AX_CARRY_LO, AX_CARRY_HI = 16, 32, 48, 64
OUTPUT_LO, OUTPUT_HI = 80, 96


def _decode_and_update(buf, g):
    rows = lax.iota(jnp.int32, L) + g * L

    def col(c):
        cols = jnp.full((L,), c, jnp.int32)
        return plsc.load_gather(buf, [rows, cols])

    mark = col(MARK_AX) > 0.5
    f_and = (col(OP_AND) > 0.5) & mark
    f_or = (col(OP_OR) > 0.5) & mark
    f_xor = (col(OP_XOR) > 0.5) & mark

    def argmax16(base):
        def body(c, carry):
            m, idx = carry
            v = col(base + c)
            gt = v > m
            return jnp.where(gt, v, m), jnp.where(gt, c, idx)

        _, idx = lax.fori_loop(
            1, 16, body, (col(base), jnp.zeros((L,), jnp.int32))
        )
        return idx

    a = argmax16(ALU_LO) + 16 * argmax16(ALU_HI)
    b = argmax16(AX_CARRY_LO) + 16 * argmax16(AX_CARRY_HI)

    res = jnp.where(f_and, a & b, 0)
    res = jnp.where(f_or, a | b, res)
    res = jnp.where(f_xor, a ^ b, res)
    active = f_and | f_or | f_xor
    vals = jnp.where(active, 1.0, 0.0).astype(jnp.float32)

    col_lo = OUTPUT_LO + (res & 15)
    col_hi = OUTPUT_HI + (res >> 4)
    plsc.addupdate_scatter(buf, [rows, col_lo], vals)
    plsc.addupdate_scatter(buf, [rows, col_hi], vals)


@functools.partial(
    pl.kernel,
    out_type=jax.ShapeDtypeStruct((B, DB), jnp.float32),
    mesh=plsc.VectorSubcoreMesh(
        core_axis_name="c", subcore_axis_name="s", num_cores=NC, num_subcores=NS
    ),
    scratch_types=[pltpu.VMEM((ROWS_PER_W, DB), jnp.float32)],
    compiler_params=pltpu.CompilerParams(needs_layout_passes=False),
)
def _sc_decode(x_hbm, blk_hbm, buf):
    wid = lax.axis_index("s") * NC + lax.axis_index("c")
    base = wid * ROWS_PER_W
    pltpu.sync_copy(x_hbm.at[pl.ds(base, ROWS_PER_W), pl.ds(0, DB)], buf)

    def group_body(g, carry):
        _decode_and_update(buf, g)
        return carry

    lax.fori_loop(0, ROWS_PER_W // L, group_body, 0)
    pltpu.sync_copy(buf, blk_hbm.at[pl.ds(base, ROWS_PER_W)])


_COPY_ROWS = 2048


def _tc_copy_body(x_ref, out_ref):
    out_ref[...] = x_ref[...]


_tc_copy = pl.pallas_call(
    _tc_copy_body,
    out_shape=jax.ShapeDtypeStruct((B, D), jnp.float32),
    grid=(B // _COPY_ROWS,),
    in_specs=[pl.BlockSpec((_COPY_ROWS, D), lambda i: (i, 0))],
    out_specs=pl.BlockSpec((_COPY_ROWS, D), lambda i: (i, 0)),
)


def _tc_merge_body(part_ref, blk_ref, out_ref):
    del part_ref
    out_ref[...] = blk_ref[...]


_MERGE_ROWS = 4096

_tc_merge = pl.pallas_call(
    _tc_merge_body,
    out_shape=jax.ShapeDtypeStruct((B, D), jnp.float32),
    grid=(B // _MERGE_ROWS,),
    in_specs=[
        pl.BlockSpec(memory_space=pl.ANY),
        pl.BlockSpec((_MERGE_ROWS, DB), lambda i: (i, 0)),
    ],
    out_specs=pl.BlockSpec((_MERGE_ROWS, DB), lambda i: (i, 0)),
    input_output_aliases={0: 0},
)


def kernel(x):
    blk = _sc_decode(x)
    part = _tc_copy(x)
    return _tc_merge(part, blk)


# speedup vs baseline: 1.0153x; 1.0153x over previous
import functools

import jax
import jax.numpy as jnp
from jax import lax
from jax.experimental import pallas as pl
from jax.experimental.pallas import tpu as pltpu
from jax.experimental.pallas import tpu_sc as plsc

B = 8192
D = 1024
DB = 128
NC = 2
NS = 16
L = 16
NW = NC * NS
ROWS_PER_W = B // NW

MARK_AX, OP_AND, OP_OR, OP_XOR = 0, 1, 2, 3
ALU_LO, ALU_HI, AX_CARRY_LO, AX_CARRY_HI = 16, 32, 48, 64
OUTPUT_LO, OUTPUT_HI = 80, 96


def _decode_and_update(buf, g):
    rows = lax.iota(jnp.int32, L) + g * L

    def col(c):
        cols = jnp.full((L,), c, jnp.int32)
        return plsc.load_gather(buf, [rows, cols])

    mark = col(MARK_AX) > 0.5
    f_and = (col(OP_AND) > 0.5) & mark
    f_or = (col(OP_OR) > 0.5) & mark
    f_xor = (col(OP_XOR) > 0.5) & mark

    def argmax16(base):
        m = col(base)
        idx = jnp.zeros((L,), jnp.int32)
        for c in range(1, 16):
            v = col(base + c)
            gt = v > m
            idx = jnp.where(gt, c, idx)
            m = jnp.where(gt, v, m)
        return idx

    a = argmax16(ALU_LO) + 16 * argmax16(ALU_HI)
    b = argmax16(AX_CARRY_LO) + 16 * argmax16(AX_CARRY_HI)

    res = jnp.where(f_and, a & b, 0)
    res = jnp.where(f_or, a | b, res)
    res = jnp.where(f_xor, a ^ b, res)
    active = f_and | f_or | f_xor
    vals = jnp.where(active, 1.0, 0.0).astype(jnp.float32)

    col_lo = OUTPUT_LO + (res & 15)
    col_hi = OUTPUT_HI + (res >> 4)
    plsc.addupdate_scatter(buf, [rows, col_lo], vals)
    plsc.addupdate_scatter(buf, [rows, col_hi], vals)


@functools.partial(
    pl.kernel,
    out_type=jax.ShapeDtypeStruct((B, DB), jnp.float32),
    mesh=plsc.VectorSubcoreMesh(
        core_axis_name="c", subcore_axis_name="s", num_cores=NC, num_subcores=NS
    ),
    scratch_types=[pltpu.VMEM((ROWS_PER_W, DB), jnp.float32)],
    compiler_params=pltpu.CompilerParams(needs_layout_passes=False),
)
def _sc_decode(x_hbm, blk_hbm, buf):
    wid = lax.axis_index("s") * NC + lax.axis_index("c")
    base = wid * ROWS_PER_W
    pltpu.sync_copy(x_hbm.at[pl.ds(base, ROWS_PER_W), pl.ds(0, DB)], buf)

    def group_body(g, carry):
        _decode_and_update(buf, g)
        return carry

    lax.fori_loop(0, ROWS_PER_W // L, group_body, 0)
    pltpu.sync_copy(buf, blk_hbm.at[pl.ds(base, ROWS_PER_W)])


_COPY_ROWS = 2048


def _tc_copy_body(x_ref, out_ref):
    out_ref[...] = x_ref[...]


_tc_copy = pl.pallas_call(
    _tc_copy_body,
    out_shape=jax.ShapeDtypeStruct((B, D), jnp.float32),
    grid=(B // _COPY_ROWS,),
    in_specs=[pl.BlockSpec((_COPY_ROWS, D), lambda i: (i, 0))],
    out_specs=pl.BlockSpec((_COPY_ROWS, D), lambda i: (i, 0)),
)


def _tc_merge_body(part_ref, blk_ref, out_ref):
    del part_ref
    out_ref[...] = blk_ref[...]


_MERGE_ROWS = 8192

_tc_merge = pl.pallas_call(
    _tc_merge_body,
    out_shape=jax.ShapeDtypeStruct((B, D), jnp.float32),
    grid=(B // _MERGE_ROWS,),
    in_specs=[
        pl.BlockSpec(memory_space=pl.ANY),
        pl.BlockSpec((_MERGE_ROWS, DB), lambda i: (i, 0)),
    ],
    out_specs=pl.BlockSpec((_MERGE_ROWS, DB), lambda i: (i, 0)),
    input_output_aliases={0: 0},
)


def kernel(x):
    blk = _sc_decode(x)
    part = _tc_copy(x)
    return _tc_merge(part, blk)


# speedup vs baseline: 1.0390x; 1.0234x over previous
import functools

import jax
import jax.numpy as jnp
from jax import lax
from jax.experimental import pallas as pl
from jax.experimental.pallas import tpu as pltpu
from jax.experimental.pallas import tpu_sc as plsc

B = 8192
D = 1024
DB = 128
NC = 2
NS = 16
L = 16
NW = NC * NS
ROWS_PER_W = B // NW

MARK_AX, OP_AND, OP_OR, OP_XOR = 0, 1, 2, 3
ALU_LO, ALU_HI, AX_CARRY_LO, AX_CARRY_HI = 16, 32, 48, 64
OUTPUT_LO, OUTPUT_HI = 80, 96


def _decode_and_update(buf, g):
    rows = lax.iota(jnp.int32, L) + g * L

    def col(c):
        cols = jnp.full((L,), c, jnp.int32)
        return plsc.load_gather(buf, [rows, cols])

    mark = col(MARK_AX) > 0.5
    f_and = (col(OP_AND) > 0.5) & mark
    f_or = (col(OP_OR) > 0.5) & mark
    f_xor = (col(OP_XOR) > 0.5) & mark

    def argmax16(base):
        m = col(base)
        idx = jnp.zeros((L,), jnp.int32)
        for c in range(1, 16):
            v = col(base + c)
            gt = v > m
            idx = jnp.where(gt, c, idx)
            m = jnp.where(gt, v, m)
        return idx

    a = argmax16(ALU_LO) + 16 * argmax16(ALU_HI)
    b = argmax16(AX_CARRY_LO) + 16 * argmax16(AX_CARRY_HI)

    res = jnp.where(f_and, a & b, 0)
    res = jnp.where(f_or, a | b, res)
    res = jnp.where(f_xor, a ^ b, res)
    active = f_and | f_or | f_xor
    vals = jnp.where(active, 1.0, 0.0).astype(jnp.float32)

    col_lo = OUTPUT_LO + (res & 15)
    col_hi = OUTPUT_HI + (res >> 4)
    plsc.addupdate_scatter(buf, [rows, col_lo], vals)
    plsc.addupdate_scatter(buf, [rows, col_hi], vals)


@functools.partial(
    pl.kernel,
    out_type=jax.ShapeDtypeStruct((B, DB), jnp.float32),
    mesh=plsc.VectorSubcoreMesh(
        core_axis_name="c", subcore_axis_name="s", num_cores=NC, num_subcores=NS
    ),
    scratch_types=[pltpu.VMEM((ROWS_PER_W, DB), jnp.float32)],
    compiler_params=pltpu.CompilerParams(needs_layout_passes=False),
)
def _sc_decode(x_hbm, blk_hbm, buf):
    wid = lax.axis_index("s") * NC + lax.axis_index("c")
    base = wid * ROWS_PER_W
    pltpu.sync_copy(x_hbm.at[pl.ds(base, ROWS_PER_W), pl.ds(0, DB)], buf)

    def group_body(g, carry):
        _decode_and_update(buf, g)
        return carry

    lax.fori_loop(0, ROWS_PER_W // L, group_body, 0)
    pltpu.sync_copy(buf, blk_hbm.at[pl.ds(base, ROWS_PER_W)])


_COPY_ROWS = 2048


def _tc_copy_body(x_ref, out_ref):
    out_ref[...] = x_ref[...]


_tc_copy = pl.pallas_call(
    _tc_copy_body,
    out_shape=jax.ShapeDtypeStruct((B, D), jnp.float32),
    grid=(B // _COPY_ROWS,),
    in_specs=[pl.BlockSpec((_COPY_ROWS, D), lambda i: (i, 0))],
    out_specs=pl.BlockSpec((_COPY_ROWS, D), lambda i: (i, 0)),
)


def _tc_merge_body(part_ref, blk_ref, out_ref):
    del part_ref
    out_ref[...] = blk_ref[...]


_MERGE_ROWS = 4096

_tc_merge = pl.pallas_call(
    _tc_merge_body,
    out_shape=jax.ShapeDtypeStruct((B, D), jnp.float32),
    grid=(B // _MERGE_ROWS,),
    in_specs=[
        pl.BlockSpec(memory_space=pl.ANY),
        pl.BlockSpec((_MERGE_ROWS, DB), lambda i: (i, 0)),
    ],
    out_specs=pl.BlockSpec((_MERGE_ROWS, DB), lambda i: (i, 0)),
    input_output_aliases={0: 0},
)


def kernel(x):
    blk = _sc_decode(x)
    part = _tc_copy(x)
    return _tc_merge(part, blk)
